# baseline (speedup 1.0000x reference)
import functools

import jax
import jax.numpy as jnp
from jax import lax
from jax.experimental import pallas as pl
from jax.experimental.pallas import tpu as pltpu
from jax.experimental.pallas import tpu_sc as plsc

D_MODEL = 1024
D_FF = 4096
N_EXPERTS = 8
T = 2048
TILE = 512
NROWS = 2 * T + N_EXPERTS * TILE
NTILES = NROWS // TILE
F_BLK = 512
NJ = D_FF // F_BLK


def _gate_body(x_ref, wg_ref, route_ref):
    l = jnp.dot(x_ref[...], wg_ref[...], preferred_element_type=jnp.float32)
    lane = lax.broadcasted_iota(jnp.int32, l.shape, 1)
    valid = lane < N_EXPERTS
    l = jnp.where(valid, l, -1e30)
    m = jnp.max(l, axis=1, keepdims=True)
    p = jnp.exp(l - m)
    p = jnp.where(valid, p, 0.0)
    probs = p / jnp.sum(p, axis=1, keepdims=True)
    m1 = jnp.max(probs, axis=1, keepdims=True)
    i1 = jnp.min(jnp.where(probs == m1, lane, 128), axis=1, keepdims=True)
    pr2 = jnp.where(lane == i1, -1.0, probs)
    m2 = jnp.max(pr2, axis=1, keepdims=True)
    i2 = jnp.min(jnp.where(pr2 == m2, lane, 128), axis=1, keepdims=True)
    den = m1 + m2 + 1e-9
    g1 = m1 / den
    g2 = m2 / den
    route_ref[...] = (
        jnp.where(lane == 0, i1.astype(jnp.float32), 0.0)
        + jnp.where(lane == 1, i2.astype(jnp.float32), 0.0)
        + jnp.where(lane == 2, g1, 0.0)
        + jnp.where(lane == 3, g2, 0.0)
    )


def _ffn_body(te_ref, used_ref, gate_ref, xs_ref, w1_ref, w2_ref, o_ref):
    i = pl.program_id(0)
    used = used_ref[i] == 1

    @pl.when(used)
    def _():
        g = gate_ref[...][:, 0:1]
        xb = xs_ref[...].astype(jnp.bfloat16)
        nq = 4
        fq = D_FF // nq
        acc = None
        for q in range(nq):
            hq = jnp.dot(xb, w1_ref[0][:, q * fq:(q + 1) * fq],
                         preferred_element_type=jnp.float32)
            aq = 0.5 * hq * (1.0 + lax.erf(hq * 0.7071067811865476)) * g
            yq = jnp.dot(aq.astype(jnp.bfloat16), w2_ref[0][q * fq:(q + 1) * fq, :],
                         preferred_element_type=jnp.float32)
            acc = yq if acc is None else acc + yq
        o_ref[...] = acc


def _add_body(y_ref, o_ref):
    o_ref[...] = y_ref[:T, :] + y_ref[T:, :]


def _make_sc_gather(n_rows, d, rows_per_chunk):
    info = plsc.get_sparse_core_info()
    nw = info.num_cores * info.num_subcores
    b_per_w = n_rows // nw
    nchunks = b_per_w // rows_per_chunk
    mesh = plsc.VectorSubcoreMesh(core_axis_name="c", subcore_axis_name="s")

    @functools.partial(
        pl.kernel,
        out_type=jax.ShapeDtypeStruct((n_rows, d), jnp.float32),
        mesh=mesh,
        scratch_types=[
            pltpu.VMEM((rows_per_chunk,), jnp.int32),
            pltpu.VMEM((rows_per_chunk, d), jnp.float32),
            pltpu.SemaphoreType.DMA,
        ],
    )
    def gather_k(table_hbm, idx_hbm, out_hbm, idx_v, rows_v, sem):
        wid = lax.axis_index("s") * info.num_cores + lax.axis_index("c")
        base = wid * b_per_w
        for c in range(nchunks):
            off = base + c * rows_per_chunk
            pltpu.sync_copy(idx_hbm.at[pl.ds(off, rows_per_chunk)], idx_v)
            pltpu.async_copy(table_hbm.at[idx_v], rows_v, sem).wait()
            pltpu.sync_copy(rows_v, out_hbm.at[pl.ds(off, rows_per_chunk)])

    return gather_k


def _make_sc_scatter(n_src, d, rows_per_chunk, n_out):
    info = plsc.get_sparse_core_info()
    nw = info.num_cores * info.num_subcores
    b_per_w = n_src // nw
    nchunks = b_per_w // rows_per_chunk
    mesh = plsc.VectorSubcoreMesh(core_axis_name="c", subcore_axis_name="s")

    @functools.partial(
        pl.kernel,
        out_type=jax.ShapeDtypeStruct((n_out, d), jnp.float32),
        mesh=mesh,
        scratch_types=[
            pltpu.VMEM((rows_per_chunk,), jnp.int32),
            pltpu.VMEM((rows_per_chunk, d), jnp.float32),
            pltpu.SemaphoreType.DMA,
        ],
    )
    def scatter_k(table_hbm, idx_hbm, out_hbm, idx_v, rows_v, sem):
        wid = lax.axis_index("s") * info.num_cores + lax.axis_index("c")
        base = wid * b_per_w
        for c in range(nchunks):
            off = base + c * rows_per_chunk
            src = off % T
            pltpu.sync_copy(table_hbm.at[pl.ds(src, rows_per_chunk)], rows_v)
            pltpu.sync_copy(idx_hbm.at[pl.ds(off, rows_per_chunk)], idx_v)
            pltpu.async_copy(rows_v, out_hbm.at[idx_v], sem).wait()

    return scatter_k


def kernel(x, wg, w1, w2):
    wg_p = jnp.zeros((D_MODEL, 128), jnp.float32).at[:, :N_EXPERTS].set(wg)
    route = pl.pallas_call(
        _gate_body,
        out_shape=jax.ShapeDtypeStruct((T, 128), jnp.float32),
    )(x, wg_p)
    e1 = route[:, 0].astype(jnp.int32)
    e2 = route[:, 1].astype(jnp.int32)
    flat_e = jnp.concatenate([e1, e2])
    gate_flat = jnp.concatenate([route[:, 2], route[:, 3]])

    counts = jnp.bincount(flat_e, length=N_EXPERTS).astype(jnp.int32)
    padded = ((counts + TILE - 1) // TILE) * TILE
    seg_end = jnp.cumsum(padded)
    seg_start = seg_end - padded
    raw_end = jnp.cumsum(counts)
    raw_start = raw_end - counts
    order = jnp.argsort(flat_e, stable=True).astype(jnp.int32)
    sorted_e = flat_e[order]
    dest_sorted = seg_start[sorted_e] + (
        jnp.arange(2 * T, dtype=jnp.int32) - raw_start[sorted_e])
    dest = jnp.zeros((2 * T,), jnp.int32).at[order].set(dest_sorted)

    tstart = jnp.arange(NTILES, dtype=jnp.int32) * TILE
    e_t = jnp.clip(jnp.searchsorted(seg_end, tstart, side="right"), 0,
                   N_EXPERTS - 1).astype(jnp.int32)
    tile_used = ((tstart - seg_start[e_t]) < counts[e_t]).astype(jnp.int32)
    ti = jnp.arange(NTILES, dtype=jnp.int32)
    last_used = lax.cummax(jnp.where(tile_used == 1, ti, -1), axis=0)
    tile_e = e_t[jnp.clip(last_used, 0, NTILES - 1)]
    gval = jnp.zeros((NROWS,), jnp.float32).at[dest].set(gate_flat)
    gate_s = jnp.broadcast_to(gval[:, None], (NROWS, 128))

    xs = _make_sc_scatter(2 * T, D_MODEL, 64, NROWS)(x, dest)

    w1b = w1.astype(jnp.bfloat16)
    w2b = w2.astype(jnp.bfloat16)
    grid_spec = pltpu.PrefetchScalarGridSpec(
        num_scalar_prefetch=2,
        grid=(NTILES,),
        in_specs=[
            pl.BlockSpec((TILE, 128), lambda i, te, u: (i, 0)),
            pl.BlockSpec((TILE, D_MODEL), lambda i, te, u: (i, 0)),
            pl.BlockSpec((1, D_MODEL, D_FF), lambda i, te, u: (te[i], 0, 0)),
            pl.BlockSpec((1, D_FF, D_MODEL), lambda i, te, u: (te[i], 0, 0)),
        ],
        out_specs=pl.BlockSpec((TILE, D_MODEL), lambda i, te, u: (i, 0)),
    )
    out_s = pl.pallas_call(
        _ffn_body,
        grid_spec=grid_spec,
        out_shape=jax.ShapeDtypeStruct((NROWS, D_MODEL), jnp.float32),
    )(tile_e, tile_used, gate_s, xs, w1b, w2b)

    yk = _make_sc_gather(2 * T, D_MODEL, 64)(out_s, dest)
    out = pl.pallas_call(
        _add_body,
        out_shape=jax.ShapeDtypeStruct((T, D_MODEL), jnp.float32),
    )(yk)
    return out


# speedup vs baseline: 1.0189x; 1.0189x over previous
import functools

import jax
import jax.numpy as jnp
from jax import lax
from jax.experimental import pallas as pl
from jax.experimental.pallas import tpu as pltpu
from jax.experimental.pallas import tpu_sc as plsc

D_MODEL = 1024
D_FF = 4096
N_EXPERTS = 8
T = 2048
TILE = 512
NROWS = 2 * T + N_EXPERTS * TILE
NTILES = NROWS // TILE
F_BLK = 512
NJ = D_FF // F_BLK


def _gate_body(x_ref, wg_ref, route_ref):
    l = jnp.dot(x_ref[...], wg_ref[...], preferred_element_type=jnp.float32)
    lane = lax.broadcasted_iota(jnp.int32, l.shape, 1)
    valid = lane < N_EXPERTS
    l = jnp.where(valid, l, -1e30)
    m = jnp.max(l, axis=1, keepdims=True)
    p = jnp.exp(l - m)
    p = jnp.where(valid, p, 0.0)
    probs = p / jnp.sum(p, axis=1, keepdims=True)
    m1 = jnp.max(probs, axis=1, keepdims=True)
    i1 = jnp.min(jnp.where(probs == m1, lane, 128), axis=1, keepdims=True)
    pr2 = jnp.where(lane == i1, -1.0, probs)
    m2 = jnp.max(pr2, axis=1, keepdims=True)
    i2 = jnp.min(jnp.where(pr2 == m2, lane, 128), axis=1, keepdims=True)
    den = m1 + m2 + 1e-9
    g1 = m1 / den
    g2 = m2 / den
    route_ref[...] = (
        jnp.where(lane == 0, i1.astype(jnp.float32), 0.0)
        + jnp.where(lane == 1, i2.astype(jnp.float32), 0.0)
        + jnp.where(lane == 2, g1, 0.0)
        + jnp.where(lane == 3, g2, 0.0)
    )


def _ffn_body(te_ref, used_ref, gate_ref, xs_ref, w1_ref, w2_ref, o_ref):
    i = pl.program_id(0)
    used = used_ref[i] == i

    @pl.when(used)
    def _():
        g = gate_ref[...][:, 0:1]
        xb = xs_ref[...].astype(jnp.bfloat16)
        nq = 4
        fq = D_FF // nq
        acc = None
        for q in range(nq):
            hq = jnp.dot(xb, w1_ref[0][:, q * fq:(q + 1) * fq],
                         preferred_element_type=jnp.float32)
            aq = 0.5 * hq * (1.0 + lax.erf(hq * 0.7071067811865476)) * g
            yq = jnp.dot(aq.astype(jnp.bfloat16), w2_ref[0][q * fq:(q + 1) * fq, :],
                         preferred_element_type=jnp.float32)
            acc = yq if acc is None else acc + yq
        o_ref[...] = acc


def _add_body(y_ref, o_ref):
    o_ref[...] = y_ref[:T, :] + y_ref[T:, :]


def _make_sc_gather(n_rows, d, rows_per_chunk):
    info = plsc.get_sparse_core_info()
    nw = info.num_cores * info.num_subcores
    b_per_w = n_rows // nw
    nchunks = b_per_w // rows_per_chunk
    mesh = plsc.VectorSubcoreMesh(core_axis_name="c", subcore_axis_name="s")

    @functools.partial(
        pl.kernel,
        out_type=jax.ShapeDtypeStruct((n_rows, d), jnp.float32),
        mesh=mesh,
        scratch_types=[
            pltpu.VMEM((rows_per_chunk,), jnp.int32),
            pltpu.VMEM((rows_per_chunk, d), jnp.float32),
            pltpu.SemaphoreType.DMA,
        ],
    )
    def gather_k(table_hbm, idx_hbm, out_hbm, idx_v, rows_v, sem):
        wid = lax.axis_index("s") * info.num_cores + lax.axis_index("c")
        base = wid * b_per_w
        for c in range(nchunks):
            off = base + c * rows_per_chunk
            pltpu.sync_copy(idx_hbm.at[pl.ds(off, rows_per_chunk)], idx_v)
            pltpu.async_copy(table_hbm.at[idx_v], rows_v, sem).wait()
            pltpu.sync_copy(rows_v, out_hbm.at[pl.ds(off, rows_per_chunk)])

    return gather_k


def _make_sc_scatter(n_src, d, rows_per_chunk, n_out):
    info = plsc.get_sparse_core_info()
    nw = info.num_cores * info.num_subcores
    b_per_w = n_src // nw
    nchunks = b_per_w // rows_per_chunk
    mesh = plsc.VectorSubcoreMesh(core_axis_name="c", subcore_axis_name="s")

    @functools.partial(
        pl.kernel,
        out_type=jax.ShapeDtypeStruct((n_out, d), jnp.float32),
        mesh=mesh,
        scratch_types=[
            pltpu.VMEM((rows_per_chunk,), jnp.int32),
            pltpu.VMEM((rows_per_chunk, d), jnp.float32),
            pltpu.SemaphoreType.DMA,
        ],
    )
    def scatter_k(table_hbm, idx_hbm, out_hbm, idx_v, rows_v, sem):
        wid = lax.axis_index("s") * info.num_cores + lax.axis_index("c")
        base = wid * b_per_w
        for c in range(nchunks):
            off = base + c * rows_per_chunk
            src = off % T
            pltpu.sync_copy(table_hbm.at[pl.ds(src, rows_per_chunk)], rows_v)
            pltpu.sync_copy(idx_hbm.at[pl.ds(off, rows_per_chunk)], idx_v)
            pltpu.async_copy(rows_v, out_hbm.at[idx_v], sem).wait()

    return scatter_k


def kernel(x, wg, w1, w2):
    wg_p = jnp.zeros((D_MODEL, 128), jnp.float32).at[:, :N_EXPERTS].set(wg)
    route = pl.pallas_call(
        _gate_body,
        out_shape=jax.ShapeDtypeStruct((T, 128), jnp.float32),
    )(x, wg_p)
    e1 = route[:, 0].astype(jnp.int32)
    e2 = route[:, 1].astype(jnp.int32)
    flat_e = jnp.concatenate([e1, e2])
    gate_flat = jnp.concatenate([route[:, 2], route[:, 3]])

    counts = jnp.bincount(flat_e, length=N_EXPERTS).astype(jnp.int32)
    padded = ((counts + TILE - 1) // TILE) * TILE
    seg_end = jnp.cumsum(padded)
    seg_start = seg_end - padded
    raw_end = jnp.cumsum(counts)
    raw_start = raw_end - counts
    order = jnp.argsort(flat_e, stable=True).astype(jnp.int32)
    sorted_e = flat_e[order]
    dest_sorted = seg_start[sorted_e] + (
        jnp.arange(2 * T, dtype=jnp.int32) - raw_start[sorted_e])
    dest = jnp.zeros((2 * T,), jnp.int32).at[order].set(dest_sorted)

    tstart = jnp.arange(NTILES, dtype=jnp.int32) * TILE
    e_t = jnp.clip(jnp.searchsorted(seg_end, tstart, side="right"), 0,
                   N_EXPERTS - 1).astype(jnp.int32)
    tile_used = ((tstart - seg_start[e_t]) < counts[e_t]).astype(jnp.int32)
    ti = jnp.arange(NTILES, dtype=jnp.int32)
    last_used = lax.cummax(jnp.where(tile_used == 1, ti, -1), axis=0)
    tile_u = jnp.clip(last_used, 0, NTILES - 1)
    tile_e = e_t[tile_u]
    gval = jnp.zeros((NROWS,), jnp.float32).at[dest].set(gate_flat)
    gate_s = jnp.broadcast_to(gval[:, None], (NROWS, 128))

    xs = _make_sc_scatter(2 * T, D_MODEL, 64, NROWS)(x, dest)

    w1b = w1.astype(jnp.bfloat16)
    w2b = w2.astype(jnp.bfloat16)
    grid_spec = pltpu.PrefetchScalarGridSpec(
        num_scalar_prefetch=2,
        grid=(NTILES,),
        in_specs=[
            pl.BlockSpec((TILE, 128), lambda i, te, u: (u[i], 0)),
            pl.BlockSpec((TILE, D_MODEL), lambda i, te, u: (u[i], 0)),
            pl.BlockSpec((1, D_MODEL, D_FF), lambda i, te, u: (te[i], 0, 0)),
            pl.BlockSpec((1, D_FF, D_MODEL), lambda i, te, u: (te[i], 0, 0)),
        ],
        out_specs=pl.BlockSpec((TILE, D_MODEL), lambda i, te, u: (u[i], 0)),
    )
    out_s = pl.pallas_call(
        _ffn_body,
        grid_spec=grid_spec,
        out_shape=jax.ShapeDtypeStruct((NROWS, D_MODEL), jnp.float32),
    )(tile_e, tile_u, gate_s, xs, w1b, w2b)

    yk = _make_sc_gather(2 * T, D_MODEL, 64)(out_s, dest)
    out = pl.pallas_call(
        _add_body,
        out_shape=jax.ShapeDtypeStruct((T, D_MODEL), jnp.float32),
    )(yk)
    return out


# speedup vs baseline: 1.1123x; 1.0917x over previous
import functools

import jax
import jax.numpy as jnp
from jax import lax
from jax.experimental import pallas as pl
from jax.experimental.pallas import tpu as pltpu
from jax.experimental.pallas import tpu_sc as plsc

D_MODEL = 1024
D_FF = 4096
N_EXPERTS = 8
T = 2048
TILE = 512
NROWS = 2 * T + N_EXPERTS * TILE
NTILES = NROWS // TILE
F_BLK = 512
NJ = D_FF // F_BLK


def _gate_body(x_ref, wg_ref, route_ref):
    l = jnp.dot(x_ref[...], wg_ref[...], preferred_element_type=jnp.float32)
    lane = lax.broadcasted_iota(jnp.int32, l.shape, 1)
    valid = lane < N_EXPERTS
    l = jnp.where(valid, l, -1e30)
    m = jnp.max(l, axis=1, keepdims=True)
    p = jnp.exp(l - m)
    p = jnp.where(valid, p, 0.0)
    probs = p / jnp.sum(p, axis=1, keepdims=True)
    m1 = jnp.max(probs, axis=1, keepdims=True)
    i1 = jnp.min(jnp.where(probs == m1, lane, 128), axis=1, keepdims=True)
    pr2 = jnp.where(lane == i1, -1.0, probs)
    m2 = jnp.max(pr2, axis=1, keepdims=True)
    i2 = jnp.min(jnp.where(pr2 == m2, lane, 128), axis=1, keepdims=True)
    den = m1 + m2 + 1e-9
    g1 = m1 / den
    g2 = m2 / den
    route_ref[...] = (
        jnp.where(lane == 0, i1.astype(jnp.float32), 0.0)
        + jnp.where(lane == 1, i2.astype(jnp.float32), 0.0)
        + jnp.where(lane == 2, g1, 0.0)
        + jnp.where(lane == 3, g2, 0.0)
    )


def _ffn_body(te_ref, used_ref, gate_ref, xs_ref, w1_ref, w2_ref, o_ref):
    i = pl.program_id(0)
    used = used_ref[i] == i

    @pl.when(used)
    def _():
        g = gate_ref[...][:, 0:1]
        xb = xs_ref[...].astype(jnp.bfloat16)
        nq = 4
        fq = D_FF // nq
        acc = None
        for q in range(nq):
            hq = jnp.dot(xb, w1_ref[0][:, q * fq:(q + 1) * fq],
                         preferred_element_type=jnp.float32)
            aq = 0.5 * hq * (1.0 + lax.erf(hq * 0.7071067811865476)) * g
            yq = jnp.dot(aq.astype(jnp.bfloat16), w2_ref[0][q * fq:(q + 1) * fq, :],
                         preferred_element_type=jnp.float32)
            acc = yq if acc is None else acc + yq
        o_ref[...] = acc


def _add_body(y_ref, o_ref):
    o_ref[...] = y_ref[:T, :] + y_ref[T:, :]


def _make_sc_gather(n_rows, d, rows_per_chunk, dtype=jnp.float32):
    info = plsc.get_sparse_core_info()
    nw = info.num_cores * info.num_subcores
    b_per_w = n_rows // nw
    nchunks = b_per_w // rows_per_chunk
    mesh = plsc.VectorSubcoreMesh(core_axis_name="c", subcore_axis_name="s")

    @functools.partial(
        pl.kernel,
        out_type=jax.ShapeDtypeStruct((n_rows, d), dtype),
        mesh=mesh,
        scratch_types=[
            pltpu.VMEM((rows_per_chunk,), jnp.int32),
            pltpu.VMEM((rows_per_chunk, d), dtype),
            pltpu.SemaphoreType.DMA,
        ],
    )
    def gather_k(table_hbm, idx_hbm, out_hbm, idx_v, rows_v, sem):
        wid = lax.axis_index("s") * info.num_cores + lax.axis_index("c")
        base = wid * b_per_w
        for c in range(nchunks):
            off = base + c * rows_per_chunk
            pltpu.sync_copy(idx_hbm.at[pl.ds(off, rows_per_chunk)], idx_v)
            pltpu.async_copy(table_hbm.at[idx_v], rows_v, sem).wait()
            pltpu.sync_copy(rows_v, out_hbm.at[pl.ds(off, rows_per_chunk)])

    return gather_k


def _make_sc_scatter(n_src, d, rows_per_chunk, n_out):
    info = plsc.get_sparse_core_info()
    nw = info.num_cores * info.num_subcores
    b_per_w = n_src // nw
    nchunks = b_per_w // rows_per_chunk
    mesh = plsc.VectorSubcoreMesh(core_axis_name="c", subcore_axis_name="s")

    @functools.partial(
        pl.kernel,
        out_type=jax.ShapeDtypeStruct((n_out, d), jnp.float32),
        mesh=mesh,
        scratch_types=[
            pltpu.VMEM((rows_per_chunk,), jnp.int32),
            pltpu.VMEM((rows_per_chunk, d), jnp.float32),
            pltpu.SemaphoreType.DMA,
        ],
    )
    def scatter_k(table_hbm, idx_hbm, out_hbm, idx_v, rows_v, sem):
        wid = lax.axis_index("s") * info.num_cores + lax.axis_index("c")
        base = wid * b_per_w
        for c in range(nchunks):
            off = base + c * rows_per_chunk
            src = off % T
            pltpu.sync_copy(table_hbm.at[pl.ds(src, rows_per_chunk)], rows_v)
            pltpu.sync_copy(idx_hbm.at[pl.ds(off, rows_per_chunk)], idx_v)
            pltpu.async_copy(rows_v, out_hbm.at[idx_v], sem).wait()

    return scatter_k


def kernel(x, wg, w1, w2):
    wg_p = jnp.zeros((D_MODEL, 128), jnp.float32).at[:, :N_EXPERTS].set(wg)
    route = pl.pallas_call(
        _gate_body,
        out_shape=jax.ShapeDtypeStruct((T, 128), jnp.float32),
    )(x, wg_p)
    e1 = route[:, 0].astype(jnp.int32)
    e2 = route[:, 1].astype(jnp.int32)
    flat_e = jnp.concatenate([e1, e2])
    gate_flat = jnp.concatenate([route[:, 2], route[:, 3]])

    onehot = (flat_e[:, None] == jnp.arange(N_EXPERTS, dtype=jnp.int32)[None, :]
              ).astype(jnp.int32)
    csum = jnp.cumsum(onehot, axis=0)
    counts = csum[-1]
    padded = ((counts + TILE - 1) // TILE) * TILE
    seg_end = jnp.cumsum(padded)
    seg_start = seg_end - padded
    rank = jnp.take_along_axis(csum, flat_e[:, None], axis=1)[:, 0] - 1
    dest = seg_start[flat_e] + rank

    tstart = jnp.arange(NTILES, dtype=jnp.int32) * TILE
    e_t = jnp.clip(jnp.searchsorted(seg_end, tstart, side="right"), 0,
                   N_EXPERTS - 1).astype(jnp.int32)
    tile_used = ((tstart - seg_start[e_t]) < counts[e_t]).astype(jnp.int32)
    ti = jnp.arange(NTILES, dtype=jnp.int32)
    last_used = lax.cummax(jnp.where(tile_used == 1, ti, -1), axis=0)
    tile_u = jnp.clip(last_used, 0, NTILES - 1)
    tile_e = e_t[tile_u]
    gval = jnp.zeros((NROWS,), jnp.float32).at[dest].set(gate_flat)
    gate_s = jnp.broadcast_to(gval[:, None], (NROWS, 128))

    xs = _make_sc_scatter(2 * T, D_MODEL, 64, NROWS)(x, dest)

    w1b = w1.astype(jnp.bfloat16)
    w2b = w2.astype(jnp.bfloat16)
    grid_spec = pltpu.PrefetchScalarGridSpec(
        num_scalar_prefetch=2,
        grid=(NTILES,),
        in_specs=[
            pl.BlockSpec((TILE, 128), lambda i, te, u: (u[i], 0)),
            pl.BlockSpec((TILE, D_MODEL), lambda i, te, u: (u[i], 0)),
            pl.BlockSpec((1, D_MODEL, D_FF), lambda i, te, u: (te[i], 0, 0)),
            pl.BlockSpec((1, D_FF, D_MODEL), lambda i, te, u: (te[i], 0, 0)),
        ],
        out_specs=pl.BlockSpec((TILE, D_MODEL), lambda i, te, u: (u[i], 0)),
    )
    out_s = pl.pallas_call(
        _ffn_body,
        grid_spec=grid_spec,
        out_shape=jax.ShapeDtypeStruct((NROWS, D_MODEL), jnp.float32),
    )(tile_e, tile_u, gate_s, xs, w1b, w2b)

    yk = _make_sc_gather(2 * T, D_MODEL, 64)(out_s, dest)
    out = pl.pallas_call(
        _add_body,
        out_shape=jax.ShapeDtypeStruct((T, D_MODEL), jnp.float32),
    )(yk)
    return out
